# baseline (speedup 1.0000x reference)
import jax
import jax.numpy as jnp
from jax import lax
from jax.experimental import pallas as pl
from jax.experimental.pallas import tpu as pltpu
from jax.experimental.pallas import tpu_sc as plsc

B = 16384
NS = 4
HB = B // NS
C = 20
D = 64
NW = 32
BPW = HB // NW
NB = 32
NCHUNK = BPW // NB
RPC = NB * C
IDXW = 80
IDXR = RPC // IDXW
NG = RPC // 16
TBH = 128


def _sc_gather(ctx_i, tgt_i, ctx_table, tgt_table, out_c, out_t,
               raw_c, raw_t, idx_c, idx_t, rows_c, rows_t, sem):
    wid = lax.axis_index("s") * 2 + lax.axis_index("c")

    def chunk(ch, carry):
        base = pl.multiple_of(wid * BPW + ch * NB, 8)
        pltpu.sync_copy(ctx_i.at[pl.ds(base, NB)], raw_c)
        pltpu.sync_copy(tgt_i.at[pl.ds(base, NB)], raw_t)

        def rbody(g, carry):
            p = g * 16 + lax.iota(jnp.int32, 16)
            b, c = p % NB, p // NB
            q8, r8 = p // IDXW, p % IDXW
            plsc.store_scatter(idx_c, [q8, r8], plsc.load_gather(raw_c, [b, c]))
            plsc.store_scatter(idx_t, [q8, r8], plsc.load_gather(raw_t, [b, c]))
            return carry

        lax.fori_loop(0, NG, rbody, 0)

        copies = []
        for j in range(IDXR):
            copies.append(pltpu.async_copy(
                ctx_table.at[idx_c.at[j]], rows_c.at[pl.ds(j * IDXW, IDXW)], sem))
            copies.append(pltpu.async_copy(
                tgt_table.at[idx_t.at[j]], rows_t.at[pl.ds(j * IDXW, IDXW)], sem))
        for cp in copies:
            cp.wait()

        outs = []
        for c in range(C):
            dst = pl.multiple_of(c * HB + base, 8)
            outs.append(pltpu.async_copy(
                rows_c.at[pl.ds(c * NB, NB)], out_c.at[pl.ds(dst, NB)], sem))
            outs.append(pltpu.async_copy(
                rows_t.at[pl.ds(c * NB, NB)], out_t.at[pl.ds(dst, NB)], sem))
        for cp in outs:
            cp.wait()
        return carry

    lax.fori_loop(0, NCHUNK, chunk, 0)


def _renorm(v):
    l2 = lax.broadcasted_iota(jnp.int32, (2 * D, 2), 0)
    k2 = lax.broadcasted_iota(jnp.int32, (2 * D, 2), 1)
    ind = jnp.where(l2 >= D, 1, 0)
    e = (ind == k2).astype(jnp.float32)
    v2 = (v * v).reshape(C * TBH, 2 * D)
    nsq = lax.dot(v2, e)
    s = jnp.where(nsq > 1.0, lax.rsqrt(nsq), 1.0)
    sfull = lax.dot(s, e.T).reshape(C, TBH, 2 * D)
    return v * sfull


def _tc_body(ctx_ref, tgt_ref, oute_ref, outo_ref):
    xs = _renorm(ctx_ref[...])
    ys = _renorm(tgt_ref[...])
    ce = jnp.sum(xs, axis=0) * (1.0 / C)
    p = ce[None, :, :] * ys
    oute_ref[...] = jnp.sum(p[:, :, :D], axis=2)
    outo_ref[...] = jnp.sum(p[:, :, D:], axis=2)


@jax.jit
def _run(ctx_i, tgt_i, ctx_table, tgt_table):
    mesh = plsc.VectorSubcoreMesh(core_axis_name="c", subcore_axis_name="s")
    gather = pl.kernel(
        _sc_gather,
        mesh=mesh,
        compiler_params=pltpu.CompilerParams(use_tc_tiling_on_sc=False,
                                             needs_layout_passes=False),
        out_type=(jax.ShapeDtypeStruct((C * HB, D), jnp.float32),
                  jax.ShapeDtypeStruct((C * HB, D), jnp.float32)),
        scratch_types=[
            pltpu.VMEM((NB, C), jnp.int32),
            pltpu.VMEM((NB, C), jnp.int32),
            pltpu.VMEM((IDXR, IDXW), jnp.int32),
            pltpu.VMEM((IDXR, IDXW), jnp.int32),
            pltpu.VMEM((RPC, D), jnp.float32),
            pltpu.VMEM((RPC, D), jnp.float32),
            pltpu.SemaphoreType.DMA,
        ],
    )
    dense = pl.pallas_call(
        _tc_body,
        grid=(HB // (2 * TBH),),
        in_specs=[pl.BlockSpec((C, TBH, 2 * D), lambda i: (0, i, 0)),
                  pl.BlockSpec((C, TBH, 2 * D), lambda i: (0, i, 0))],
        out_specs=[pl.BlockSpec((C, TBH), lambda i: (0, i)),
                   pl.BlockSpec((C, TBH), lambda i: (0, i))],
        out_shape=[jax.ShapeDtypeStruct((C, HB // 2), jnp.float32),
                   jax.ShapeDtypeStruct((C, HB // 2), jnp.float32)],
    )

    sims = []
    for h in range(NS):
        ci = lax.slice_in_dim(ctx_i, h * HB, (h + 1) * HB, axis=0)
        ti = lax.slice_in_dim(tgt_i, h * HB, (h + 1) * HB, axis=0)
        rows_c, rows_t = gather(ci, ti, ctx_table, tgt_table)
        rows_c = rows_c.reshape(C, HB // 2, 2 * D)
        rows_t = rows_t.reshape(C, HB // 2, 2 * D)
        sims.append(dense(rows_c, rows_t))
    halves = [jnp.stack([se.T, so.T], axis=1).reshape(HB, C)
              for se, so in sims]
    return lax.concatenate(halves, 0)


def kernel(contexts, targets, ctx_table, tgt_table):
    return _run(contexts, targets, ctx_table, tgt_table)


# speedup vs baseline: 1.3429x; 1.3429x over previous
import jax
import jax.numpy as jnp
from jax import lax
from jax.experimental import pallas as pl
from jax.experimental.pallas import tpu as pltpu
from jax.experimental.pallas import tpu_sc as plsc

B = 16384
NS = 4
HB = B // NS
C = 20
D = 64
NW = 32
BPW = HB // NW
NB = 32
NCHUNK = BPW // NB
RPC = NB * C
IDXW = 80
IDXR = RPC // IDXW
NG = RPC // 16
TBH = 128


def _sc_gather(ctx_i, tgt_i, ctx_table, tgt_table, out_c, out_t,
               raw_c, raw_t, idx_c, idx_t, rows_c, rows_t, sem):
    wid = lax.axis_index("s") * 2 + lax.axis_index("c")

    def chunk(ch, carry):
        base = pl.multiple_of(wid * BPW + ch * NB, 8)
        pltpu.sync_copy(ctx_i.at[pl.ds(base, NB)], raw_c)
        pltpu.sync_copy(tgt_i.at[pl.ds(base, NB)], raw_t)

        def rbody(g, carry):
            p = g * 16 + lax.iota(jnp.int32, 16)
            b, c = p % NB, p // NB
            q8, r8 = p // IDXW, p % IDXW
            plsc.store_scatter(idx_c, [q8, r8], plsc.load_gather(raw_c, [b, c]))
            plsc.store_scatter(idx_t, [q8, r8], plsc.load_gather(raw_t, [b, c]))
            return carry

        lax.fori_loop(0, NG, rbody, 0)

        copies = []
        for j in range(IDXR):
            copies.append(pltpu.async_copy(
                ctx_table.at[idx_c.at[j]], rows_c.at[pl.ds(j * IDXW, IDXW)], sem))
            copies.append(pltpu.async_copy(
                tgt_table.at[idx_t.at[j]], rows_t.at[pl.ds(j * IDXW, IDXW)], sem))
        for cp in copies:
            cp.wait()

        outs = []
        for c in range(C):
            dst = pl.multiple_of(c * HB + base, 8)
            outs.append(pltpu.async_copy(
                rows_c.at[pl.ds(c * NB, NB)], out_c.at[pl.ds(dst, NB)], sem))
            outs.append(pltpu.async_copy(
                rows_t.at[pl.ds(c * NB, NB)], out_t.at[pl.ds(dst, NB)], sem))
        for cp in outs:
            cp.wait()
        return carry

    lax.fori_loop(0, NCHUNK, chunk, 0)


def _renorm(v):
    nsq = jnp.sum(v * v, axis=2, keepdims=True)
    return v * jnp.where(nsq > 1.0, lax.rsqrt(nsq), 1.0)


def _tc_body(ctx_ref, tgt_ref, oute_ref, outo_ref):
    x = ctx_ref[...]
    xe = _renorm(x[:, :, :D])
    xo = _renorm(x[:, :, D:])
    cee = jnp.sum(xe, axis=0) * (1.0 / C)
    ceo = jnp.sum(xo, axis=0) * (1.0 / C)
    y = tgt_ref[...]
    ye = _renorm(y[:, :, :D])
    yo = _renorm(y[:, :, D:])
    oute_ref[...] = jnp.sum(cee[None, :, :] * ye, axis=2)
    outo_ref[...] = jnp.sum(ceo[None, :, :] * yo, axis=2)


@jax.jit
def _run(ctx_i, tgt_i, ctx_table, tgt_table):
    mesh = plsc.VectorSubcoreMesh(core_axis_name="c", subcore_axis_name="s")
    gather = pl.kernel(
        _sc_gather,
        mesh=mesh,
        compiler_params=pltpu.CompilerParams(use_tc_tiling_on_sc=False,
                                             needs_layout_passes=False),
        out_type=(jax.ShapeDtypeStruct((C * HB, D), jnp.float32),
                  jax.ShapeDtypeStruct((C * HB, D), jnp.float32)),
        scratch_types=[
            pltpu.VMEM((NB, C), jnp.int32),
            pltpu.VMEM((NB, C), jnp.int32),
            pltpu.VMEM((IDXR, IDXW), jnp.int32),
            pltpu.VMEM((IDXR, IDXW), jnp.int32),
            pltpu.VMEM((RPC, D), jnp.float32),
            pltpu.VMEM((RPC, D), jnp.float32),
            pltpu.SemaphoreType.DMA,
        ],
    )
    dense = pl.pallas_call(
        _tc_body,
        grid=(HB // (2 * TBH),),
        in_specs=[pl.BlockSpec((C, TBH, 2 * D), lambda i: (0, i, 0)),
                  pl.BlockSpec((C, TBH, 2 * D), lambda i: (0, i, 0))],
        out_specs=[pl.BlockSpec((C, TBH), lambda i: (0, i)),
                   pl.BlockSpec((C, TBH), lambda i: (0, i))],
        out_shape=[jax.ShapeDtypeStruct((C, HB // 2), jnp.float32),
                   jax.ShapeDtypeStruct((C, HB // 2), jnp.float32)],
    )

    sims = []
    for h in range(NS):
        ci = lax.slice_in_dim(ctx_i, h * HB, (h + 1) * HB, axis=0)
        ti = lax.slice_in_dim(tgt_i, h * HB, (h + 1) * HB, axis=0)
        rows_c, rows_t = gather(ci, ti, ctx_table, tgt_table)
        rows_c = rows_c.reshape(C, HB // 2, 2 * D)
        rows_t = rows_t.reshape(C, HB // 2, 2 * D)
        sims.append(dense(rows_c, rows_t))
    halves = [jnp.stack([se.T, so.T], axis=1).reshape(HB, C)
              for se, so in sims]
    return lax.concatenate(halves, 0)


def kernel(contexts, targets, ctx_table, tgt_table):
    return _run(contexts, targets, ctx_table, tgt_table)
